# baseline (speedup 1.0000x reference)
import functools

import jax
import jax.numpy as jnp
from jax import lax
from jax.experimental import pallas as pl
from jax.experimental.pallas import tpu as pltpu
from jax.experimental.pallas import tpu_sc as plsc

NC = 2
NS = 16
NW = NC * NS
L = 16
C = 8
DEPTH = 4
G = 4


def _extract(vec_ref, j):
    grp = (j // L) * L
    vec = vec_ref[pl.ds(grp, L)]
    onehot = lax.iota(jnp.int32, L) == (j - grp)
    return jnp.sum(jnp.where(onehot, vec, 0))


@functools.partial(jax.jit, static_argnums=(5, 6))
def _sc_masked_add(x2, a2, gidx, gidx2, starts, N, D):
    RW = N // NW
    nchunk = RW // C

    def body(x_hbm, a_hbm, gidx_hbm, gidx2_hbm, starts_hbm, out_hbm,
             xgb, agb, idxm, idxv, stv,
             cs0, cs1, cs2, cs3, gx0, gx1, ga0, ga1, ws0, ws1):
        csems = (cs0, cs1, cs2, cs3)
        gxs = (gx0, gx1)
        gas = (ga0, ga1)
        wss = (ws0, ws1)
        cid = lax.axis_index("c")
        sid = lax.axis_index("s")
        w = sid * NC + cid
        base = w * RW
        pltpu.sync_copy(gidx_hbm.at[w], idxv)
        pltpu.sync_copy(gidx2_hbm.at[w], idxm)
        pltpu.sync_copy(starts_hbm.at[w], stv)
        kw = _extract(stv, nchunk)
        ng = (kw + (G - 1)) // G

        def issue_chunk(c, sem):
            pltpu.async_copy(x_hbm.at[pl.ds(base + c * C, C)],
                             out_hbm.at[pl.ds(base + c * C, C)], sem)

        for sl in range(DEPTH):
            issue_chunk(sl, csems[sl])

        def drain_one(nc):
            for i in range(DEPTH):
                @pl.when((nc % DEPTH) == i)
                def _():
                    pltpu.make_async_copy(
                        x_hbm.at[pl.ds(base, C)],
                        out_hbm.at[pl.ds(base, C)], csems[i]).wait()

                    @pl.when(nc + DEPTH < nchunk)
                    def _():
                        issue_chunk(nc + DEPTH, csems[i])
            return nc + 1

        def group_work(grp, t, nc):
            @pl.when(grp >= 2)
            def _():
                pltpu.make_async_copy(
                    x_hbm.at[pl.ds(base, G)],
                    out_hbm.at[pl.ds(base, G)], wss[t]).wait()
            pltpu.async_copy(x_hbm.at[idxm.at[grp]], xgb.at[t], gxs[t])
            pltpu.async_copy(a_hbm.at[idxm.at[grp]], agb.at[t], gas[t])
            cm = (_extract(idxv, grp * G + G - 1) - base) // C
            nc = lax.while_loop(lambda v: v <= cm, drain_one, nc)
            pltpu.make_async_copy(x_hbm.at[pl.ds(base, G)],
                                  xgb.at[t], gxs[t]).wait()
            pltpu.make_async_copy(a_hbm.at[pl.ds(base, G)],
                                  agb.at[t], gas[t]).wait()
            for r in range(G):
                def add_body(d, c2):
                    slc = pl.ds(d * L, L)
                    plsc.addupdate(xgb.at[t, r, slc], agb[t, r, slc])
                    return c2
                lax.fori_loop(0, D // L, add_body, 0, unroll=8)
            pltpu.async_copy(xgb.at[t], out_hbm.at[idxm.at[grp]], wss[t])
            return nc

        def pair_body(pi, nc):
            for t in range(2):
                grp = pi * 2 + t
                nc = lax.cond(grp < ng,
                              functools.partial(group_work, grp, t),
                              lambda v: v, nc)
            return nc

        nc = lax.fori_loop(0, (ng + 1) // 2, pair_body, 0)
        nc = lax.while_loop(lambda v: v < nchunk, drain_one, nc)
        for t in range(2):
            @pl.when(ng > t)
            def _():
                pltpu.make_async_copy(x_hbm.at[pl.ds(base, G)],
                                      out_hbm.at[pl.ds(base, G)], wss[t]).wait()

    fn = pl.kernel(
        body,
        out_type=jax.ShapeDtypeStruct((N, D), jnp.float32),
        mesh=plsc.VectorSubcoreMesh(
            core_axis_name="c", subcore_axis_name="s",
            num_cores=NC, num_subcores=NS),
        scratch_types=[
            pltpu.VMEM((2, G, D), jnp.float32),
            pltpu.VMEM((2, G, D), jnp.float32),
            pltpu.VMEM((RW // G, G), jnp.int32),
            pltpu.VMEM((RW,), jnp.int32),
            pltpu.VMEM((64,), jnp.int32),
        ] + [pltpu.SemaphoreType.DMA] * 10,
        compiler_params=pltpu.CompilerParams(needs_layout_passes=False),
    )
    return fn(x2, a2, gidx, gidx2, starts)


def kernel(x, attack, attack_mask):
    B, S, D = x.shape
    N = B * S
    RW = N // NW
    x2 = x.reshape(N, D)
    a2 = attack.astype(x.dtype).reshape(N, D)
    m2 = attack_mask[:, :S].reshape(NW, RW)
    loc = jnp.argsort(~m2, axis=1, stable=True).astype(jnp.int32)
    kwv = m2.sum(axis=1).astype(jnp.int32)
    last = jnp.take_along_axis(loc, jnp.maximum(kwv - 1, 0)[:, None], axis=1)
    locp = jnp.where(jnp.arange(RW, dtype=jnp.int32)[None, :] < kwv[:, None],
                     loc, last)
    gidx = locp + (jnp.arange(NW, dtype=jnp.int32) * RW)[:, None]
    gidx2 = gidx.reshape(NW, RW // G, G)
    ccnt = m2.reshape(NW, RW // C, C).sum(-1).astype(jnp.int32)
    starts = jnp.concatenate(
        [jnp.zeros((NW, 1), jnp.int32), jnp.cumsum(ccnt, axis=1)], axis=1)
    starts = jnp.pad(starts, ((0, 0), (0, 64 - starts.shape[1])))
    out2 = _sc_masked_add(x2, a2, gidx, gidx2, starts, N, D)
    return out2.reshape(B, S, D)


# speedup vs baseline: 21.8333x; 21.8333x over previous
import functools

import jax
import jax.numpy as jnp
from jax import lax
from jax.experimental import pallas as pl
from jax.experimental.pallas import tpu as pltpu
from jax.experimental.pallas import tpu_sc as plsc

NC = 2
NS = 16
NW = NC * NS
L = 16
C = 4
R = 4


def _extract(vec_ref, j):
    grp = (j // L) * L
    vec = vec_ref[pl.ds(grp, L)]
    onehot = lax.iota(jnp.int32, L) == (j - grp)
    return jnp.sum(jnp.where(onehot, vec, 0))


@functools.partial(jax.jit, static_argnums=(4, 5))
def _sc_masked_add(x2, a2, gidx, starts, N, D):
    RW = N // NW
    nchunk = RW // C

    def body(x_hbm, a_hbm, gidx_hbm, starts_hbm, out_hbm,
             buf, abuf, idxv, stv,
             in0, in1, in2, in3, ou0, ou1, ou2, ou3, gsem):
        ins = (in0, in1, in2, in3)
        outs = (ou0, ou1, ou2, ou3)
        cid = lax.axis_index("c")
        sid = lax.axis_index("s")
        w = sid * NC + cid
        base = w * RW
        pltpu.sync_copy(gidx_hbm.at[w], idxv)
        pltpu.sync_copy(starts_hbm.at[w], stv)

        def issue_in(c, t):
            pltpu.async_copy(x_hbm.at[pl.ds(base + c * C, C)],
                             buf.at[t], ins[t])

        issue_in(0, 0)

        def chunk_work(c, u, s):
            t = u % R
            e = _extract(stv, c + 1)
            k = e - s

            def fire(i, _):
                g = _extract(idxv, s + i)
                pltpu.async_copy(a_hbm.at[pl.ds(g, 1)],
                                 abuf.at[pl.ds(i, 1)], gsem)
                return _

            lax.fori_loop(0, k, fire, 0)
            pltpu.make_async_copy(x_hbm.at[pl.ds(base, C)],
                                  buf.at[t], ins[t]).wait()
            t1 = (u + 1) % R

            @pl.when(c + 1 < nchunk)
            def _():
                @pl.when(c >= R - 1)
                def _():
                    pltpu.make_async_copy(buf.at[t1],
                                          out_hbm.at[pl.ds(base, C)],
                                          outs[t1]).wait()
                issue_in(c + 1, t1)

            def drain(i, _):
                pltpu.make_async_copy(a_hbm.at[pl.ds(base, 1)],
                                      abuf.at[pl.ds(0, 1)], gsem).wait()
                return _

            lax.fori_loop(0, k, drain, 0)

            def patch(i, _):
                g = _extract(idxv, s + i)
                p = g - (base + c * C)

                def add_body(d, c2):
                    slc = pl.ds(d * L, L)
                    plsc.addupdate(buf.at[t, p, slc], abuf[i, slc])
                    return c2

                lax.fori_loop(0, D // L, add_body, 0, unroll=8)
                return _

            lax.fori_loop(0, k, patch, 0)
            pltpu.async_copy(buf.at[t], out_hbm.at[pl.ds(base + c * C, C)],
                             outs[t])
            return e

        def group_body(gi, s):
            for u in range(R):
                s = chunk_work(gi * R + u, u, s)
            return s

        lax.fori_loop(0, nchunk // R, group_body, 0)
        for t in range(R):
            pltpu.make_async_copy(buf.at[t], out_hbm.at[pl.ds(base, C)],
                                  outs[t]).wait()

    fn = pl.kernel(
        body,
        out_type=jax.ShapeDtypeStruct((N, D), jnp.float32),
        mesh=plsc.VectorSubcoreMesh(
            core_axis_name="c", subcore_axis_name="s",
            num_cores=NC, num_subcores=NS),
        scratch_types=[
            pltpu.VMEM((R, C, D), jnp.float32),
            pltpu.VMEM((C, D), jnp.float32),
            pltpu.VMEM((RW,), jnp.int32),
            pltpu.VMEM((96,), jnp.int32),
        ] + [pltpu.SemaphoreType.DMA] * 9,
        compiler_params=pltpu.CompilerParams(needs_layout_passes=False),
    )
    return fn(x2, a2, gidx, starts)


def kernel(x, attack, attack_mask):
    B, S, D = x.shape
    N = B * S
    RW = N // NW
    x2 = x.reshape(N, D)
    a2 = attack.astype(x.dtype).reshape(N, D)
    m2 = attack_mask[:, :S].reshape(NW, RW)
    loc = jnp.argsort(~m2, axis=1, stable=True).astype(jnp.int32)
    gidx = loc + (jnp.arange(NW, dtype=jnp.int32) * RW)[:, None]
    ccnt = m2.reshape(NW, RW // C, C).sum(-1).astype(jnp.int32)
    starts = jnp.concatenate(
        [jnp.zeros((NW, 1), jnp.int32), jnp.cumsum(ccnt, axis=1)], axis=1)
    starts = jnp.pad(starts, ((0, 0), (0, 96 - starts.shape[1])))
    out2 = _sc_masked_add(x2, a2, gidx, starts, N, D)
    return out2.reshape(B, S, D)
